# Initial kernel scaffold; baseline (speedup 1.0000x reference)
import jax
import jax.numpy as jnp
from jax.experimental import pallas as pl


def kernel(src_seq, word_table, accent_table):
    raise NotImplementedError("write your pallas kernel here")


# speedup vs baseline: 1.6750x; 1.6750x over previous
import functools

import jax
import jax.numpy as jnp
from jax import lax
from jax.experimental import pallas as pl
from jax.experimental.pallas import tpu as pltpu
from jax.experimental.pallas import tpu_sc as plsc

NC = 2
NS = 16
LANES = 16
NW = NC * NS

D_WORD = 32
CHUNK = 512


@functools.cache
def _build(n_rows: int):
    assert n_rows % (NW * CHUNK) == 0
    rows_per_w = n_rows // NW
    n_chunks = rows_per_w // CHUNK
    mesh = plsc.VectorSubcoreMesh(core_axis_name="c", subcore_axis_name="s")

    @functools.partial(
        pl.kernel,
        out_type=jax.ShapeDtypeStruct((n_rows, D_WORD), jnp.float32),
        mesh=mesh,
        compiler_params=pltpu.CompilerParams(
            needs_layout_passes=False, use_tc_tiling_on_sc=False),
        scratch_types=[
            pltpu.VMEM((2 * CHUNK,), jnp.int32),
            pltpu.VMEM((CHUNK,), jnp.int32),
            pltpu.VMEM((CHUNK,), jnp.int32),
            pltpu.VMEM((CHUNK, D_WORD), jnp.float32),
            pltpu.VMEM((CHUNK, D_WORD), jnp.float32),
            pltpu.SemaphoreType.DMA,
            pltpu.SemaphoreType.DMA,
        ],
    )
    def k(src_hbm, word_hbm, accent_hbm, out_hbm,
          chunk_v, tidx_v, aidx_v, trows_v, arows_v, sem_t, sem_a):
        wid = lax.axis_index("s") * NC + lax.axis_index("c")
        row0 = wid * rows_per_w
        lane = lax.iota(jnp.int32, LANES)

        def chunk_body(g, _):
            base = row0 + g * CHUNK
            pltpu.sync_copy(src_hbm.at[pl.ds(2 * base, 2 * CHUNK)], chunk_v)

            def deint(j, _):
                off = j * LANES
                pos = (off + lane) * 2
                tidx_v[pl.ds(off, LANES)] = plsc.load_gather(chunk_v, [pos])
                aidx_v[pl.ds(off, LANES)] = plsc.load_gather(chunk_v, [pos + 1])
                return 0

            lax.fori_loop(0, CHUNK // LANES, deint, 0, unroll=4)

            ct = pltpu.async_copy(word_hbm.at[tidx_v], trows_v, sem_t)
            ca = pltpu.async_copy(accent_hbm.at[aidx_v], arows_v, sem_a)
            ct.wait()
            ca.wait()

            def addrow(r, _):
                plsc.addupdate(trows_v.at[r, pl.ds(0, LANES)],
                               arows_v[r, pl.ds(0, LANES)])
                plsc.addupdate(trows_v.at[r, pl.ds(LANES, LANES)],
                               arows_v[r, pl.ds(LANES, LANES)])
                return 0

            lax.fori_loop(0, CHUNK, addrow, 0, unroll=4)
            pltpu.sync_copy(trows_v, out_hbm.at[pl.ds(base, CHUNK)])
            return 0

        lax.fori_loop(0, n_chunks, chunk_body, 0)

    return k


def kernel(src_seq, word_table, accent_table):
    b, l = src_seq.shape
    n_rows = b * l // 2
    out = _build(n_rows)(src_seq.reshape(-1), word_table, accent_table)
    return out.reshape(b, l // 2, D_WORD)


# speedup vs baseline: 6.7725x; 4.0434x over previous
import functools

import jax
import jax.numpy as jnp
from jax import lax
from jax.experimental import pallas as pl
from jax.experimental.pallas import tpu as pltpu
from jax.experimental.pallas import tpu_sc as plsc

NC = 2
NS = 16
LANES = 16
NW = NC * NS

D_WORD = 32
BB = 128
PITCH = 33


@functools.cache
def _build(n_b: int, n_t: int):
    sh = 2 * n_t // 8
    nblk = n_b // BB
    blk_per_w = nblk // NW
    n_units = blk_per_w * n_t
    assert nblk % NW == 0 and (2 * n_t) % 8 == 0
    mesh = plsc.VectorSubcoreMesh(core_axis_name="c", subcore_axis_name="s")

    @functools.partial(
        pl.kernel,
        out_type=jax.ShapeDtypeStruct((n_t, D_WORD // 8, nblk, 8, BB),
                                      jnp.float32),
        mesh=mesh,
        compiler_params=pltpu.CompilerParams(
            needs_layout_passes=False, use_tc_tiling_on_sc=False),
        scratch_types=[
            pltpu.VMEM((sh, 8, BB), jnp.int32),
            pltpu.VMEM((2, BB, D_WORD), jnp.float32),
            pltpu.VMEM((2, BB, D_WORD), jnp.float32),
            pltpu.VMEM((BB * PITCH,), jnp.float32),
            pltpu.VMEM((2, D_WORD // 8, 8, BB), jnp.float32),
            pltpu.SemaphoreType.DMA((2,)),
            pltpu.SemaphoreType.DMA((2,)),
        ],
    )
    def k(src4_hbm, word_hbm, accent_hbm, out_hbm,
          idx_v, trows_v, arows_v, sum_v, obuf_v, sem_g, sem_o):
        wid = lax.axis_index("s") * NC + lax.axis_index("c")
        lane = lax.iota(jnp.int32, LANES)
        lane_pitch = lane * PITCH

        def issue(t, slot):
            j = (2 * t) // 8
            kk = (2 * t) % 8
            pltpu.async_copy(word_hbm.at[idx_v.at[j, kk]],
                             trows_v.at[slot], sem_g.at[slot])
            pltpu.async_copy(accent_hbm.at[idx_v.at[j, kk + 1]],
                             arows_v.at[slot], sem_g.at[slot])

        def drain_gathers(slot):
            pltpu.make_async_copy(word_hbm.at[idx_v.at[0, 0]],
                                  trows_v.at[slot], sem_g.at[slot]).wait()
            pltpu.make_async_copy(accent_hbm.at[idx_v.at[0, 0]],
                                  arows_v.at[slot], sem_g.at[slot]).wait()

        def wait_out(slot):
            pltpu.make_async_copy(obuf_v.at[slot], out_hbm.at[0, :, 0],
                                  sem_o.at[slot]).wait()

        def unit(u, _):
            t = u % n_t
            bi = u // n_t
            slot = u % 2
            blk = wid * blk_per_w + bi

            @pl.when(t == 0)
            def _():
                def stage(j, _):
                    pltpu.sync_copy(src4_hbm.at[j, blk], idx_v.at[j])
                    return 0
                lax.fori_loop(0, sh, stage, 0)
                issue(0, slot)

            @pl.when(t < n_t - 1)
            def _():
                issue(t + 1, 1 - slot)

            drain_gathers(slot)

            def addrow(r, _):
                t0 = trows_v[slot, r, pl.ds(0, LANES)]
                t1 = trows_v[slot, r, pl.ds(LANES, LANES)]
                a0 = arows_v[slot, r, pl.ds(0, LANES)]
                a1 = arows_v[slot, r, pl.ds(LANES, LANES)]
                sum_v[pl.ds(r * PITCH, LANES)] = t0 + a0
                sum_v[pl.ds(r * PITCH + LANES, LANES)] = t1 + a1
                return 0
            lax.fori_loop(0, BB, addrow, 0, unroll=4)

            @pl.when(u > 1)
            def _():
                wait_out(slot)

            def tcol(a, _):
                def tsub(cs, _):
                    c = a * 8 + cs

                    def tvec(q, _):
                        idx = lane_pitch + (c + q * (LANES * PITCH))
                        obuf_v[slot, a, cs, pl.ds(q * LANES, LANES)] = (
                            plsc.load_gather(sum_v, [idx]))
                        return 0
                    lax.fori_loop(0, BB // LANES, tvec, 0, unroll=8)
                    return 0
                lax.fori_loop(0, 8, tsub, 0)
                return 0
            lax.fori_loop(0, D_WORD // 8, tcol, 0)

            pltpu.async_copy(obuf_v.at[slot], out_hbm.at[t, :, blk],
                             sem_o.at[slot])
            return 0

        lax.fori_loop(0, n_units, unit, 0)
        wait_out(0)
        wait_out(1)

    return k


def kernel(src_seq, word_table, accent_table):
    b, l = src_seq.shape
    n_t = l // 2
    src4 = (src_seq.T.reshape(l // 8, 8, b // BB, BB)
            .transpose(0, 2, 1, 3))
    out5 = _build(b, n_t)(src4, word_table, accent_table)
    return (out5.transpose(2, 4, 0, 1, 3)
            .reshape(b, n_t, D_WORD))


# speedup vs baseline: 7.0211x; 1.0367x over previous
import functools

import jax
import jax.numpy as jnp
from jax import lax
from jax.experimental import pallas as pl
from jax.experimental.pallas import tpu as pltpu
from jax.experimental.pallas import tpu_sc as plsc

NC = 2
NS = 16
LANES = 16
NW = NC * NS

D_WORD = 32
BB = 128
PITCH = 33


@functools.cache
def _build(n_b: int, n_t: int):
    sh = 2 * n_t // 8
    nblk = n_b // BB
    blk_per_w = nblk // NW
    n_units = blk_per_w * n_t
    assert nblk % NW == 0 and (2 * n_t) % 8 == 0
    mesh = plsc.VectorSubcoreMesh(core_axis_name="c", subcore_axis_name="s")

    @functools.partial(
        pl.kernel,
        out_type=jax.ShapeDtypeStruct((n_t, D_WORD // 8, nblk, 8, BB),
                                      jnp.float32),
        mesh=mesh,
        compiler_params=pltpu.CompilerParams(
            needs_layout_passes=False, use_tc_tiling_on_sc=False),
        scratch_types=[
            pltpu.VMEM((sh, 8, BB), jnp.int32),
            pltpu.VMEM((4, BB, D_WORD), jnp.float32),
            pltpu.VMEM((4, BB, D_WORD), jnp.float32),
            pltpu.VMEM((BB * PITCH,), jnp.float32),
            pltpu.VMEM((2, D_WORD // 8, 8, BB), jnp.float32),
            pltpu.SemaphoreType.DMA((4,)),
            pltpu.SemaphoreType.DMA((2,)),
            pltpu.SemaphoreType.DMA,
        ],
    )
    def k(src4_hbm, word_hbm, accent_hbm, out_hbm,
          idx_v, trows_v, arows_v, sum_v, obuf_v, sem_g, sem_o, sem_i):
        wid = lax.axis_index("s") * NC + lax.axis_index("c")
        lane = lax.iota(jnp.int32, LANES)
        lane_pitch = lane * PITCH
        DEPTH = 4

        def issue(t, slot):
            j = (2 * t) // 8
            kk = (2 * t) % 8
            pltpu.async_copy(word_hbm.at[idx_v.at[j, kk]],
                             trows_v.at[slot], sem_g.at[slot])
            pltpu.async_copy(accent_hbm.at[idx_v.at[j, kk + 1]],
                             arows_v.at[slot], sem_g.at[slot])

        def drain_gathers(slot):
            pltpu.make_async_copy(word_hbm.at[idx_v.at[0, 0]],
                                  trows_v.at[slot], sem_g.at[slot]).wait()
            pltpu.make_async_copy(accent_hbm.at[idx_v.at[0, 0]],
                                  arows_v.at[slot], sem_g.at[slot]).wait()

        def wait_out(slot):
            pltpu.make_async_copy(obuf_v.at[slot], out_hbm.at[0, :, 0],
                                  sem_o.at[slot]).wait()

        def block_body(bi, _):
            blk = wid * blk_per_w + bi

            def stage(j, _):
                pltpu.async_copy(src4_hbm.at[j, blk], idx_v.at[j], sem_i)
                return 0
            lax.fori_loop(0, sh, stage, 0)

            def stage_wait(j, _):
                pltpu.make_async_copy(src4_hbm.at[0, 0], idx_v.at[0],
                                      sem_i).wait()
                return 0
            lax.fori_loop(0, sh, stage_wait, 0)

            for t0 in range(DEPTH - 1):
                issue(t0, t0)

            def unit(t, _):
                slot = t % DEPTH

                @pl.when(t < n_t - (DEPTH - 1))
                def _():
                    issue(t + DEPTH - 1, (t + DEPTH - 1) % DEPTH)

                drain_gathers(slot)

                def addrow(r, off):
                    t0 = trows_v[slot, r, pl.ds(0, LANES)]
                    t1 = trows_v[slot, r, pl.ds(LANES, LANES)]
                    a0 = arows_v[slot, r, pl.ds(0, LANES)]
                    a1 = arows_v[slot, r, pl.ds(LANES, LANES)]
                    sum_v[pl.ds(off, LANES)] = t0 + a0
                    sum_v[pl.ds(off + LANES, LANES)] = t1 + a1
                    return off + PITCH
                lax.fori_loop(0, BB, addrow, 0, unroll=8)

                oslot = t % 2
                @pl.when((bi > 0) | (t > 1))
                def _():
                    wait_out(oslot)

                def tcol(a, _):
                    def tsub(cs, _):
                        c = a * 8 + cs
                        base = lane_pitch + c
                        for q in range(BB // LANES):
                            obuf_v[oslot, a, cs,
                                   pl.ds(q * LANES, LANES)] = (
                                plsc.load_gather(
                                    sum_v, [base + q * (LANES * PITCH)]))
                        return 0
                    lax.fori_loop(0, 8, tsub, 0)
                    return 0
                lax.fori_loop(0, D_WORD // 8, tcol, 0)

                pltpu.async_copy(obuf_v.at[oslot], out_hbm.at[t, :, blk],
                                 sem_o.at[oslot])
                return 0

            lax.fori_loop(0, n_t, unit, 0)
            return 0

        lax.fori_loop(0, blk_per_w, block_body, 0)
        wait_out(0)
        wait_out(1)

    return k


def kernel(src_seq, word_table, accent_table):
    b, l = src_seq.shape
    n_t = l // 2
    src4 = (src_seq.T.reshape(l // 8, 8, b // BB, BB)
            .transpose(0, 2, 1, 3))
    n_v = word_table.shape[0]
    wt = lax.optimization_barrier(
        word_table.reshape(n_v // 4, 4 * D_WORD)).reshape(n_v, D_WORD)
    at = lax.optimization_barrier(
        accent_table.reshape(n_v // 4, 4 * D_WORD)).reshape(n_v, D_WORD)
    out5 = _build(b, n_t)(src4, wt, at)
    return (out5.transpose(2, 4, 0, 1, 3)
            .reshape(b, n_t, D_WORD))
